# baseline (speedup 1.0000x reference)
import functools

import jax
import jax.numpy as jnp
from jax import lax
from jax.experimental import pallas as pl
from jax.experimental.pallas import tpu as pltpu
from jax.experimental.pallas import tpu_sc as plsc

NUM_CORES = 2
NUM_SUBCORES = 16
NUM_WORKERS = NUM_CORES * NUM_SUBCORES
BATCH = 16384
EMBED_DIM = 64
B_PER_W = BATCH // NUM_WORKERS
LANES = 16

_mesh = plsc.VectorSubcoreMesh(core_axis_name="c", subcore_axis_name="s")


@functools.partial(
    pl.kernel,
    mesh=_mesh,
    compiler_params=pltpu.CompilerParams(use_tc_tiling_on_sc=False),
    out_type=[
        jax.ShapeDtypeStruct((BATCH,), jnp.float32),
        jax.ShapeDtypeStruct((BATCH,), jnp.float32),
    ],
    scratch_types=[
        pltpu.VMEM((B_PER_W,), jnp.int32),
        pltpu.VMEM((B_PER_W,), jnp.int32),
        pltpu.VMEM((B_PER_W,), jnp.int32),
        pltpu.VMEM((B_PER_W, EMBED_DIM), jnp.float32),
        pltpu.VMEM((B_PER_W, EMBED_DIM), jnp.float32),
        pltpu.VMEM((B_PER_W, EMBED_DIM), jnp.float32),
        pltpu.VMEM((B_PER_W,), jnp.float32),
        pltpu.VMEM((B_PER_W,), jnp.float32),
        pltpu.SemaphoreType.DMA,
    ],
)
def _sc_fused(u_tab, i_tab, uid, pid, nid, pos_out, neg_out,
              uidx, pidx, nidx, urows, prows, nrows, pos_v, neg_v, sem):
    wid = lax.axis_index("s") * NUM_CORES + lax.axis_index("c")
    base = wid * B_PER_W
    sl = pl.ds(base, B_PER_W)
    pltpu.sync_copy(uid.at[sl], uidx)
    pltpu.sync_copy(pid.at[sl], pidx)
    pltpu.sync_copy(nid.at[sl], nidx)
    cu = pltpu.async_copy(u_tab.at[uidx], urows, sem)
    cp = pltpu.async_copy(i_tab.at[pidx], prows, sem)
    cn = pltpu.async_copy(i_tab.at[nidx], nrows, sem)
    cu.wait()
    cp.wait()
    cn.wait()

    lane = lax.iota(jnp.int32, LANES)
    perms = [lane ^ k for k in (8, 4, 2, 1)]

    dnums = lax.GatherDimensionNumbers(
        offset_dims=(), collapsed_slice_dims=(0,), start_index_map=(0,))

    def lane_sum(v):
        for p in perms:
            shuf = lax.gather(v, p[:, None], dnums, (1,),
                              mode=lax.GatherScatterMode.PROMISE_IN_BOUNDS)
            v = v + shuf
        return v

    @pl.loop(0, B_PER_W // LANES)
    def _(g):
        accp = jnp.zeros((LANES,), jnp.float32)
        accn = jnp.zeros((LANES,), jnp.float32)
        for j in range(LANES):
            i = g * LANES + j
            sp = jnp.zeros((LANES,), jnp.float32)
            sn = jnp.zeros((LANES,), jnp.float32)
            for c in range(EMBED_DIM // LANES):
                u = urows[i, pl.ds(c * LANES, LANES)]
                sp = sp + u * prows[i, pl.ds(c * LANES, LANES)]
                sn = sn + u * nrows[i, pl.ds(c * LANES, LANES)]
            mask = lane == j
            accp = jnp.where(mask, lane_sum(sp), accp)
            accn = jnp.where(mask, lane_sum(sn), accn)
        pos_v[pl.ds(g * LANES, LANES)] = accp
        neg_v[pl.ds(g * LANES, LANES)] = accn

    wp = pltpu.async_copy(pos_v, pos_out.at[sl], sem)
    wn = pltpu.async_copy(neg_v, neg_out.at[sl], sem)
    wp.wait()
    wn.wait()


def kernel(user_ids, pos_item_ids, neg_item_ids, user_table, item_table):
    uid = user_ids.astype(jnp.int32)
    pid = pos_item_ids.astype(jnp.int32)
    nid = neg_item_ids.astype(jnp.int32)
    return tuple(_sc_fused(user_table, item_table, uid, pid, nid))
